# baseline (speedup 1.0000x reference)
import functools

import jax
import jax.numpy as jnp
from jax import lax
from jax.experimental import pallas as pl
from jax.experimental.pallas import tpu as pltpu
from jax.experimental.pallas import tpu_sc as plsc

N_CORES = 2
N_SUBCORES = 16
NW = N_CORES * N_SUBCORES
LANES = 16
CHUNK = 128
BATCH = 8
_AGG_SPLIT = (19, 1)


def _pad_edges(src, dst, sink):
    e = src.shape[0]
    unit = NW * BATCH * CHUNK
    e_pad = -(-e // unit) * unit
    if e_pad != e:
        src = jnp.concatenate(
            [src, jnp.zeros((e_pad - e,), jnp.int32)])
        dst = jnp.concatenate(
            [dst, jnp.full((e_pad - e,), sink, jnp.int32)])
    return src.reshape(-1, CHUNK), dst.reshape(-1, CHUNK)


def _acc_rows(n_nodes):
    return -(-(n_nodes + 1) // (N_SUBCORES * 128)) * (N_SUBCORES * 128)


@functools.lru_cache(maxsize=None)
def _make_agg(n_nodes, n_chunks, d, split=None):
    assert n_chunks % (NW * BATCH) == 0
    n_batches = n_chunks // (NW * BATCH)
    b0, b1 = split if split is not None else (n_batches, n_batches)
    assert b0 + b1 == 2 * n_batches
    n_acc = _acc_rows(n_nodes)
    rps = n_acc // N_SUBCORES
    zb = 32
    assert rps % zb == 0

    mesh = plsc.VectorSubcoreMesh(core_axis_name="c", subcore_axis_name="s")

    out_type = jax.ShapeDtypeStruct((N_CORES, n_acc, d), jnp.float32)
    scratch = [
        pltpu.VMEM_SHARED((n_acc, d), jnp.float32),
        pltpu.VMEM((BATCH, CHUNK), jnp.int32),
        pltpu.VMEM((BATCH, CHUNK), jnp.int32),
        pltpu.VMEM((CHUNK, d), jnp.float32),
        pltpu.VMEM((CHUNK, d), jnp.float32),
        pltpu.VMEM((zb, d), jnp.float32),
        pltpu.SemaphoreType.DMA,
        pltpu.SemaphoreType.DMA,
        pltpu.SemaphoreType.DMA,
        pltpu.SemaphoreType.DMA,
    ]

    def body(x_hbm, src_hbm, dst_hbm, sum_hbm, acc_sh, srcb_v, dstb_v,
             rows0_v, rows1_v, zrow_v, gsem0, gsem1, ssem0, ssem1):
        cid = lax.axis_index("c")
        sid = lax.axis_index("s")
        wid = cid * N_SUBCORES + sid
        r0 = sid * rps
        rows = (rows0_v, rows1_v)
        gsem = (gsem0, gsem1)
        ssem = (ssem0, ssem1)

        @pl.loop(0, zb)
        def _(r):
            @pl.loop(0, d, step=LANES)
            def _(c):
                zrow_v[r, pl.ds(c, LANES)] = jnp.zeros((LANES,), jnp.float32)

        @pl.loop(0, rps, step=zb)
        def _(rr):
            pltpu.sync_copy(zrow_v, acc_sh.at[pl.ds(r0 + rr, zb)])

        plsc.subcore_barrier()

        nb = jnp.where(cid == 0, b0, b1)
        chunk0 = jnp.where(cid == 0, sid * b0,
                           N_SUBCORES * b0 + sid * b1) * BATCH

        @pl.loop(0, nb)
        def _(t):
            cbase = chunk0 + t * BATCH
            pltpu.sync_copy(src_hbm.at[pl.ds(cbase, BATCH)], srcb_v)
            pltpu.sync_copy(dst_hbm.at[pl.ds(cbase, BATCH)], dstb_v)
            ga = [
                pltpu.async_copy(x_hbm.at[srcb_v.at[0]], rows[0], gsem[0]),
                pltpu.async_copy(x_hbm.at[srcb_v.at[1]], rows[1], gsem[1]),
            ]
            sc = [None, None]
            for j in range(BATCH):
                b = j % 2
                ga[b].wait()
                sc[b] = pltpu.async_copy(rows[b], acc_sh.at[dstb_v.at[j]],
                                         ssem[b], add=True)
                if 1 <= j < BATCH - 1:
                    sc[1 - b].wait()
                    ga[1 - b] = pltpu.async_copy(
                        x_hbm.at[srcb_v.at[j + 1]], rows[1 - b], gsem[1 - b])
            sc[0].wait()
            sc[1].wait()

        plsc.subcore_barrier()

        pltpu.sync_copy(acc_sh.at[pl.ds(r0, rps)],
                        sum_hbm.at[cid, pl.ds(r0, rps)])

    return pl.kernel(body, out_type=out_type, mesh=mesh,
                     scratch_types=scratch)


@functools.lru_cache(maxsize=None)
def _make_counts(n_nodes, n_chunks):
    assert n_chunks % (NW * BATCH) == 0
    n_batches = n_chunks // (NW * BATCH)
    n_acc = _acc_rows(n_nodes)
    rps = n_acc // N_SUBCORES

    mesh = plsc.VectorSubcoreMesh(core_axis_name="c", subcore_axis_name="s")

    out_type = jax.ShapeDtypeStruct((N_CORES * n_acc,), jnp.float32)
    scratch = [
        pltpu.VMEM_SHARED((n_acc,), jnp.float32),
        pltpu.VMEM((BATCH, CHUNK), jnp.int32),
        pltpu.VMEM((CHUNK,), jnp.float32),
        pltpu.SemaphoreType.DMA,
    ]

    def body(dst_hbm, cnt_hbm, cnt_sh, dstb_v, ones_v, sem):
        cid = lax.axis_index("c")
        sid = lax.axis_index("s")
        wid = cid * N_SUBCORES + sid
        r0 = sid * rps

        @pl.loop(0, CHUNK, step=LANES)
        def _(r):
            ones_v[pl.ds(r, LANES)] = jnp.zeros((LANES,), jnp.float32)

        @pl.loop(0, rps, step=CHUNK)
        def _(rr):
            pltpu.sync_copy(ones_v, cnt_sh.at[pl.ds(r0 + rr, CHUNK)])

        @pl.loop(0, CHUNK, step=LANES)
        def _(r):
            ones_v[pl.ds(r, LANES)] = jnp.ones((LANES,), jnp.float32)

        plsc.subcore_barrier()

        chunk0 = wid * (n_batches * BATCH)

        @pl.loop(0, n_batches)
        def _(t):
            pltpu.sync_copy(dst_hbm.at[pl.ds(chunk0 + t * BATCH, BATCH)],
                            dstb_v)
            for j in range(BATCH):
                pltpu.sync_copy(ones_v, cnt_sh.at[dstb_v.at[j]], add=True)

        plsc.subcore_barrier()
        pltpu.sync_copy(cnt_sh.at[pl.ds(r0, rps)],
                        cnt_hbm.at[pl.ds(cid * n_acc + r0, rps)])

    return pl.kernel(body, out_type=out_type, mesh=mesh,
                     scratch_types=scratch)


@functools.lru_cache(maxsize=None)
def _make_layer(n_acc, d, relu, bn):
    assert n_acc % bn == 0
    grid = (n_acc // bn,)

    def body(s_ref, c_ref, x_ref, wl_ref, wr_ref, b_ref, o_ref):
        s = s_ref[0] + s_ref[1]
        cnt = c_ref[:, 0:1] + c_ref[:, 1:2]
        mean = s / jnp.maximum(cnt, 1.0)
        acc = jnp.dot(mean, wl_ref[...], preferred_element_type=jnp.float32)
        acc = acc + jnp.dot(x_ref[...], wr_ref[...],
                            preferred_element_type=jnp.float32)
        acc = acc + b_ref[...]
        if relu:
            acc = jnp.maximum(acc, 0.0)
        o_ref[...] = acc

    return pl.pallas_call(
        body,
        grid=grid,
        in_specs=[
            pl.BlockSpec((N_CORES, bn, d), lambda i: (0, i, 0)),
            pl.BlockSpec((bn, N_CORES), lambda i: (i, 0)),
            pl.BlockSpec((bn, d), lambda i: (i, 0)),
            pl.BlockSpec((d, d), lambda i: (0, 0)),
            pl.BlockSpec((d, d), lambda i: (0, 0)),
            pl.BlockSpec((1, d), lambda i: (0, 0)),
        ],
        out_specs=pl.BlockSpec((bn, d), lambda i: (i, 0)),
        out_shape=jax.ShapeDtypeStruct((n_acc, d), jnp.float32),
    )


def kernel(x, edge_index, W1_l, b1_l, W1_r, b1_r, W2_l, b2_l, W2_r, b2_r):
    n, d = x.shape
    src = edge_index[0].astype(jnp.int32)
    dst = edge_index[1].astype(jnp.int32)
    n_acc = _acc_rows(n)
    src2d, dst2d = _pad_edges(src, dst, n)
    n_chunks = src2d.shape[0]
    x_p = jnp.pad(x, ((0, n_acc - n), (0, 0)))

    agg = _make_agg(n, n_chunks, d, _AGG_SPLIT)
    cnts = _make_counts(n, n_chunks)(dst2d)
    cpair = cnts.reshape(N_CORES, n_acc).T

    sums1 = agg(x, src2d, dst2d)
    h = _make_layer(n_acc, d, True, 1024)(
        sums1, cpair, x_p, W1_l, W1_r, (b1_l + b1_r).reshape(1, d))
    sums2 = agg(h, src2d, dst2d)
    out = _make_layer(n_acc, d, False, 1024)(
        sums2, cpair, h, W2_l, W2_r, (b2_l + b2_r).reshape(1, d))
    return out[:n]


# speedup vs baseline: 1.0184x; 1.0184x over previous
import functools

import jax
import jax.numpy as jnp
from jax import lax
from jax.experimental import pallas as pl
from jax.experimental.pallas import tpu as pltpu
from jax.experimental.pallas import tpu_sc as plsc

N_CORES = 2
N_SUBCORES = 16
NW = N_CORES * N_SUBCORES
LANES = 16
CHUNK = 128
BATCH = 16
_AGG_SPLIT = (9, 1)


def _pad_edges(src, dst, sink):
    e = src.shape[0]
    unit = NW * BATCH * CHUNK
    e_pad = -(-e // unit) * unit
    if e_pad != e:
        src = jnp.concatenate(
            [src, jnp.zeros((e_pad - e,), jnp.int32)])
        dst = jnp.concatenate(
            [dst, jnp.full((e_pad - e,), sink, jnp.int32)])
    return src.reshape(-1, CHUNK), dst.reshape(-1, CHUNK)


def _acc_rows(n_nodes):
    return -(-(n_nodes + 1) // (N_SUBCORES * 128)) * (N_SUBCORES * 128)


@functools.lru_cache(maxsize=None)
def _make_agg(n_nodes, n_chunks, d, split=None):
    assert n_chunks % (NW * BATCH) == 0
    n_batches = n_chunks // (NW * BATCH)
    b0, b1 = split if split is not None else (n_batches, n_batches)
    assert b0 + b1 == 2 * n_batches
    n_acc = _acc_rows(n_nodes)
    rps = n_acc // N_SUBCORES
    zb = 32
    assert rps % zb == 0

    mesh = plsc.VectorSubcoreMesh(core_axis_name="c", subcore_axis_name="s")

    out_type = jax.ShapeDtypeStruct((N_CORES, n_acc, d), jnp.float32)
    scratch = [
        pltpu.VMEM_SHARED((n_acc, d), jnp.float32),
        pltpu.VMEM((BATCH, CHUNK), jnp.int32),
        pltpu.VMEM((BATCH, CHUNK), jnp.int32),
        pltpu.VMEM((CHUNK, d), jnp.float32),
        pltpu.VMEM((CHUNK, d), jnp.float32),
        pltpu.VMEM((zb, d), jnp.float32),
        pltpu.SemaphoreType.DMA,
        pltpu.SemaphoreType.DMA,
        pltpu.SemaphoreType.DMA,
        pltpu.SemaphoreType.DMA,
        pltpu.SemaphoreType.DMA,
        pltpu.SemaphoreType.DMA,
    ]

    def body(x_hbm, src_hbm, dst_hbm, sum_hbm, acc_sh, srcb_v, dstb_v,
             rows0_v, rows1_v, zrow_v, gsem0, gsem1, gsem2, gsem3,
             ssem0, ssem1):
        cid = lax.axis_index("c")
        sid = lax.axis_index("s")
        wid = cid * N_SUBCORES + sid
        r0 = sid * rps
        rows = (rows0_v, rows1_v)
        gsem = ((gsem0, gsem1), (gsem2, gsem3))
        ssem = (ssem0, ssem1)
        half = CHUNK // 2

        def start_gather(idx_row, b):
            return (
                pltpu.async_copy(x_hbm.at[srcb_v.at[idx_row, pl.ds(0, half)]],
                                 rows[b].at[pl.ds(0, half)], gsem[b][0]),
                pltpu.async_copy(
                    x_hbm.at[srcb_v.at[idx_row, pl.ds(half, half)]],
                    rows[b].at[pl.ds(half, half)], gsem[b][1]),
            )

        def wait_gather(h):
            h[0].wait()
            h[1].wait()

        @pl.loop(0, zb)
        def _(r):
            @pl.loop(0, d, step=LANES)
            def _(c):
                zrow_v[r, pl.ds(c, LANES)] = jnp.zeros((LANES,), jnp.float32)

        @pl.loop(0, rps, step=zb)
        def _(rr):
            pltpu.sync_copy(zrow_v, acc_sh.at[pl.ds(r0 + rr, zb)])

        plsc.subcore_barrier()

        nb = jnp.where(cid == 0, b0, b1)
        chunk0 = jnp.where(cid == 0, sid * b0,
                           N_SUBCORES * b0 + sid * b1) * BATCH

        @pl.loop(0, nb)
        def _(t):
            cbase = chunk0 + t * BATCH
            pltpu.sync_copy(src_hbm.at[pl.ds(cbase, BATCH)], srcb_v)
            pltpu.sync_copy(dst_hbm.at[pl.ds(cbase, BATCH)], dstb_v)
            ga = [start_gather(0, 0), start_gather(1, 1)]
            sc = [None, None]
            for j in range(BATCH):
                b = j % 2
                wait_gather(ga[b])
                sc[b] = pltpu.async_copy(rows[b], acc_sh.at[dstb_v.at[j]],
                                         ssem[b], add=True)
                if 1 <= j < BATCH - 1:
                    sc[1 - b].wait()
                    ga[1 - b] = start_gather(j + 1, 1 - b)
            sc[0].wait()
            sc[1].wait()

        plsc.subcore_barrier()

        pltpu.sync_copy(acc_sh.at[pl.ds(r0, rps)],
                        sum_hbm.at[cid, pl.ds(r0, rps)])

    return pl.kernel(body, out_type=out_type, mesh=mesh,
                     scratch_types=scratch)


@functools.lru_cache(maxsize=None)
def _make_counts(n_nodes, n_chunks):
    assert n_chunks % (NW * BATCH) == 0
    n_batches = n_chunks // (NW * BATCH)
    n_acc = _acc_rows(n_nodes)
    rps = n_acc // N_SUBCORES

    mesh = plsc.VectorSubcoreMesh(core_axis_name="c", subcore_axis_name="s")

    out_type = jax.ShapeDtypeStruct((N_CORES * n_acc,), jnp.float32)
    scratch = [
        pltpu.VMEM_SHARED((n_acc,), jnp.float32),
        pltpu.VMEM((BATCH, CHUNK), jnp.int32),
        pltpu.VMEM((CHUNK,), jnp.float32),
        pltpu.SemaphoreType.DMA,
    ]

    def body(dst_hbm, cnt_hbm, cnt_sh, dstb_v, ones_v, sem):
        cid = lax.axis_index("c")
        sid = lax.axis_index("s")
        wid = cid * N_SUBCORES + sid
        r0 = sid * rps

        @pl.loop(0, CHUNK, step=LANES)
        def _(r):
            ones_v[pl.ds(r, LANES)] = jnp.zeros((LANES,), jnp.float32)

        @pl.loop(0, rps, step=CHUNK)
        def _(rr):
            pltpu.sync_copy(ones_v, cnt_sh.at[pl.ds(r0 + rr, CHUNK)])

        @pl.loop(0, CHUNK, step=LANES)
        def _(r):
            ones_v[pl.ds(r, LANES)] = jnp.ones((LANES,), jnp.float32)

        plsc.subcore_barrier()

        chunk0 = wid * (n_batches * BATCH)

        @pl.loop(0, n_batches)
        def _(t):
            pltpu.sync_copy(dst_hbm.at[pl.ds(chunk0 + t * BATCH, BATCH)],
                            dstb_v)
            for j in range(BATCH):
                pltpu.sync_copy(ones_v, cnt_sh.at[dstb_v.at[j]], add=True)

        plsc.subcore_barrier()
        pltpu.sync_copy(cnt_sh.at[pl.ds(r0, rps)],
                        cnt_hbm.at[pl.ds(cid * n_acc + r0, rps)])

    return pl.kernel(body, out_type=out_type, mesh=mesh,
                     scratch_types=scratch)


@functools.lru_cache(maxsize=None)
def _make_layer(n_acc, d, relu, bn):
    assert n_acc % bn == 0
    grid = (n_acc // bn,)

    def body(s_ref, c_ref, x_ref, wl_ref, wr_ref, b_ref, o_ref):
        s = s_ref[0] + s_ref[1]
        cnt = c_ref[:, 0:1] + c_ref[:, 1:2]
        mean = s / jnp.maximum(cnt, 1.0)
        acc = jnp.dot(mean, wl_ref[...], preferred_element_type=jnp.float32)
        acc = acc + jnp.dot(x_ref[...], wr_ref[...],
                            preferred_element_type=jnp.float32)
        acc = acc + b_ref[...]
        if relu:
            acc = jnp.maximum(acc, 0.0)
        o_ref[...] = acc

    return pl.pallas_call(
        body,
        grid=grid,
        in_specs=[
            pl.BlockSpec((N_CORES, bn, d), lambda i: (0, i, 0)),
            pl.BlockSpec((bn, N_CORES), lambda i: (i, 0)),
            pl.BlockSpec((bn, d), lambda i: (i, 0)),
            pl.BlockSpec((d, d), lambda i: (0, 0)),
            pl.BlockSpec((d, d), lambda i: (0, 0)),
            pl.BlockSpec((1, d), lambda i: (0, 0)),
        ],
        out_specs=pl.BlockSpec((bn, d), lambda i: (i, 0)),
        out_shape=jax.ShapeDtypeStruct((n_acc, d), jnp.float32),
    )


def kernel(x, edge_index, W1_l, b1_l, W1_r, b1_r, W2_l, b2_l, W2_r, b2_r):
    n, d = x.shape
    src = edge_index[0].astype(jnp.int32)
    dst = edge_index[1].astype(jnp.int32)
    n_acc = _acc_rows(n)
    src2d, dst2d = _pad_edges(src, dst, n)
    n_chunks = src2d.shape[0]
    x_p = jnp.pad(x, ((0, n_acc - n), (0, 0)))

    agg = _make_agg(n, n_chunks, d, _AGG_SPLIT)
    cnts = _make_counts(n, n_chunks)(dst2d)
    cpair = cnts.reshape(N_CORES, n_acc).T

    sums1 = agg(x, src2d, dst2d)
    h = _make_layer(n_acc, d, True, 1024)(
        sums1, cpair, x_p, W1_l, W1_r, (b1_l + b1_r).reshape(1, d))
    sums2 = agg(h, src2d, dst2d)
    out = _make_layer(n_acc, d, False, 1024)(
        sums2, cpair, h, W2_l, W2_r, (b2_l + b2_r).reshape(1, d))
    return out[:n]


# speedup vs baseline: 1.0185x; 1.0001x over previous
import functools

import jax
import jax.numpy as jnp
from jax import lax
from jax.experimental import pallas as pl
from jax.experimental.pallas import tpu as pltpu
from jax.experimental.pallas import tpu_sc as plsc

N_CORES = 2
N_SUBCORES = 16
NW = N_CORES * N_SUBCORES
LANES = 16
CHUNK = 128
BATCH = 16
_AGG_SPLIT = (9, 1)


def _pad_edges(src, dst, sink):
    e = src.shape[0]
    unit = NW * BATCH * CHUNK
    e_pad = -(-e // unit) * unit
    if e_pad != e:
        src = jnp.concatenate(
            [src, jnp.zeros((e_pad - e,), jnp.int32)])
        dst = jnp.concatenate(
            [dst, jnp.full((e_pad - e,), sink, jnp.int32)])
    return src.reshape(-1, CHUNK), dst.reshape(-1, CHUNK)


def _acc_rows(n_nodes):
    return -(-(n_nodes + 1) // (N_SUBCORES * 128)) * (N_SUBCORES * 128)


@functools.lru_cache(maxsize=None)
def _make_agg(n_nodes, n_chunks, d, split=None):
    assert n_chunks % (NW * BATCH) == 0
    n_batches = n_chunks // (NW * BATCH)
    b0, b1 = split if split is not None else (n_batches, n_batches)
    assert b0 + b1 == 2 * n_batches
    n_acc = _acc_rows(n_nodes)
    rps = n_acc // N_SUBCORES
    zb = 32
    assert rps % zb == 0

    mesh = plsc.VectorSubcoreMesh(core_axis_name="c", subcore_axis_name="s")

    out_type = jax.ShapeDtypeStruct((N_CORES, n_acc, d), jnp.float32)
    scratch = [
        pltpu.VMEM_SHARED((n_acc, d), jnp.float32),
        pltpu.VMEM((BATCH, CHUNK), jnp.int32),
        pltpu.VMEM((BATCH, CHUNK), jnp.int32),
        pltpu.VMEM((CHUNK, d), jnp.float32),
        pltpu.VMEM((CHUNK, d), jnp.float32),
        pltpu.VMEM((zb, d), jnp.float32),
        pltpu.SemaphoreType.DMA,
        pltpu.SemaphoreType.DMA,
        pltpu.SemaphoreType.DMA,
        pltpu.SemaphoreType.DMA,
    ]

    def body(x_hbm, src_hbm, dst_hbm, sum_hbm, acc_sh, srcb_v, dstb_v,
             rows0_v, rows1_v, zrow_v, gsem0, gsem1, ssem0, ssem1):
        cid = lax.axis_index("c")
        sid = lax.axis_index("s")
        wid = cid * N_SUBCORES + sid
        r0 = sid * rps
        rows = (rows0_v, rows1_v)
        gsem = (gsem0, gsem1)
        ssem = (ssem0, ssem1)

        @pl.loop(0, zb)
        def _(r):
            @pl.loop(0, d, step=LANES)
            def _(c):
                zrow_v[r, pl.ds(c, LANES)] = jnp.zeros((LANES,), jnp.float32)

        @pl.loop(0, rps, step=zb)
        def _(rr):
            pltpu.sync_copy(zrow_v, acc_sh.at[pl.ds(r0 + rr, zb)])

        plsc.subcore_barrier()

        nb = jnp.where(cid == 0, b0, b1)
        chunk0 = jnp.where(cid == 0, sid * b0,
                           N_SUBCORES * b0 + sid * b1) * BATCH

        @pl.loop(0, nb)
        def _(t):
            cbase = chunk0 + t * BATCH
            pltpu.sync_copy(src_hbm.at[pl.ds(cbase, BATCH)], srcb_v)
            pltpu.sync_copy(dst_hbm.at[pl.ds(cbase, BATCH)], dstb_v)
            ga = [
                pltpu.async_copy(x_hbm.at[srcb_v.at[0]], rows[0], gsem[0]),
                pltpu.async_copy(x_hbm.at[srcb_v.at[1]], rows[1], gsem[1]),
            ]
            sc = [None, None]
            for j in range(BATCH):
                b = j % 2
                ga[b].wait()
                sc[b] = pltpu.async_copy(rows[b], acc_sh.at[dstb_v.at[j]],
                                         ssem[b], add=True)
                if 1 <= j < BATCH - 1:
                    sc[1 - b].wait()
                    ga[1 - b] = pltpu.async_copy(
                        x_hbm.at[srcb_v.at[j + 1]], rows[1 - b], gsem[1 - b])
            sc[0].wait()
            sc[1].wait()

        plsc.subcore_barrier()

        pltpu.sync_copy(acc_sh.at[pl.ds(r0, rps)],
                        sum_hbm.at[cid, pl.ds(r0, rps)])

    return pl.kernel(body, out_type=out_type, mesh=mesh,
                     scratch_types=scratch)


@functools.lru_cache(maxsize=None)
def _make_counts(n_nodes, n_chunks):
    assert n_chunks % (NW * BATCH) == 0
    n_batches = n_chunks // (NW * BATCH)
    n_acc = _acc_rows(n_nodes)
    rps = n_acc // N_SUBCORES

    mesh = plsc.VectorSubcoreMesh(core_axis_name="c", subcore_axis_name="s")

    out_type = jax.ShapeDtypeStruct((N_CORES * n_acc,), jnp.float32)
    scratch = [
        pltpu.VMEM_SHARED((n_acc,), jnp.float32),
        pltpu.VMEM((BATCH, CHUNK), jnp.int32),
        pltpu.VMEM((CHUNK,), jnp.float32),
        pltpu.SemaphoreType.DMA,
    ]

    def body(dst_hbm, cnt_hbm, cnt_sh, dstb_v, ones_v, sem):
        cid = lax.axis_index("c")
        sid = lax.axis_index("s")
        wid = cid * N_SUBCORES + sid
        r0 = sid * rps

        @pl.loop(0, CHUNK, step=LANES)
        def _(r):
            ones_v[pl.ds(r, LANES)] = jnp.zeros((LANES,), jnp.float32)

        @pl.loop(0, rps, step=CHUNK)
        def _(rr):
            pltpu.sync_copy(ones_v, cnt_sh.at[pl.ds(r0 + rr, CHUNK)])

        @pl.loop(0, CHUNK, step=LANES)
        def _(r):
            ones_v[pl.ds(r, LANES)] = jnp.ones((LANES,), jnp.float32)

        plsc.subcore_barrier()

        chunk0 = wid * (n_batches * BATCH)

        @pl.loop(0, n_batches)
        def _(t):
            pltpu.sync_copy(dst_hbm.at[pl.ds(chunk0 + t * BATCH, BATCH)],
                            dstb_v)
            for j in range(BATCH):
                pltpu.sync_copy(ones_v, cnt_sh.at[dstb_v.at[j]], add=True)

        plsc.subcore_barrier()
        pltpu.sync_copy(cnt_sh.at[pl.ds(r0, rps)],
                        cnt_hbm.at[pl.ds(cid * n_acc + r0, rps)])

    return pl.kernel(body, out_type=out_type, mesh=mesh,
                     scratch_types=scratch)


@functools.lru_cache(maxsize=None)
def _make_layer(n_acc, d, relu, bn):
    assert n_acc % bn == 0
    grid = (n_acc // bn,)

    def body(s_ref, c_ref, x_ref, wl_ref, wr_ref, b_ref, o_ref):
        s = s_ref[0] + s_ref[1]
        cnt = c_ref[:, 0:1] + c_ref[:, 1:2]
        mean = s / jnp.maximum(cnt, 1.0)
        acc = jnp.dot(mean, wl_ref[...], preferred_element_type=jnp.float32)
        acc = acc + jnp.dot(x_ref[...], wr_ref[...],
                            preferred_element_type=jnp.float32)
        acc = acc + b_ref[...]
        if relu:
            acc = jnp.maximum(acc, 0.0)
        o_ref[...] = acc

    return pl.pallas_call(
        body,
        grid=grid,
        in_specs=[
            pl.BlockSpec((N_CORES, bn, d), lambda i: (0, i, 0)),
            pl.BlockSpec((bn, N_CORES), lambda i: (i, 0)),
            pl.BlockSpec((bn, d), lambda i: (i, 0)),
            pl.BlockSpec((d, d), lambda i: (0, 0)),
            pl.BlockSpec((d, d), lambda i: (0, 0)),
            pl.BlockSpec((1, d), lambda i: (0, 0)),
        ],
        out_specs=pl.BlockSpec((bn, d), lambda i: (i, 0)),
        out_shape=jax.ShapeDtypeStruct((n_acc, d), jnp.float32),
    )


def kernel(x, edge_index, W1_l, b1_l, W1_r, b1_r, W2_l, b2_l, W2_r, b2_r):
    n, d = x.shape
    src = edge_index[0].astype(jnp.int32)
    dst = edge_index[1].astype(jnp.int32)
    n_acc = _acc_rows(n)
    src2d, dst2d = _pad_edges(src, dst, n)
    n_chunks = src2d.shape[0]
    x_p = jnp.pad(x, ((0, n_acc - n), (0, 0)))

    agg = _make_agg(n, n_chunks, d, _AGG_SPLIT)
    cnts = _make_counts(n, n_chunks)(dst2d)
    cpair = cnts.reshape(N_CORES, n_acc).T

    sums1 = agg(x, src2d, dst2d)
    h = _make_layer(n_acc, d, True, 1024)(
        sums1, cpair, x_p, W1_l, W1_r, (b1_l + b1_r).reshape(1, d))
    sums2 = agg(h, src2d, dst2d)
    out = _make_layer(n_acc, d, False, 1024)(
        sums2, cpair, h, W2_l, W2_r, (b2_l + b2_r).reshape(1, d))
    return out[:n]


# speedup vs baseline: 1.0351x; 1.0163x over previous
import functools

import jax
import jax.numpy as jnp
from jax import lax
from jax.experimental import pallas as pl
from jax.experimental.pallas import tpu as pltpu
from jax.experimental.pallas import tpu_sc as plsc

N_CORES = 2
N_SUBCORES = 16
NW = N_CORES * N_SUBCORES
LANES = 16
CHUNK = 128
BATCH = 16
_AGG_SPLIT = (9, 1)


def _pad_edges(src, dst, sink):
    e = src.shape[0]
    unit = NW * BATCH * CHUNK
    e_pad = -(-e // unit) * unit
    if e_pad != e:
        src = jnp.concatenate(
            [src, jnp.zeros((e_pad - e,), jnp.int32)])
        dst = jnp.concatenate(
            [dst, jnp.full((e_pad - e,), sink, jnp.int32)])
    return src.reshape(-1, CHUNK), dst.reshape(-1, CHUNK)


def _acc_rows(n_nodes):
    return -(-(n_nodes + 1) // (N_SUBCORES * 128)) * (N_SUBCORES * 128)


@functools.lru_cache(maxsize=None)
def _make_agg(n_nodes, n_chunks, d, split=None, with_counts=False):
    assert n_chunks % (NW * BATCH) == 0
    n_batches = n_chunks // (NW * BATCH)
    b0, b1 = split if split is not None else (n_batches, n_batches)
    assert b0 + b1 == 2 * n_batches
    n_acc = _acc_rows(n_nodes)
    rps = n_acc // N_SUBCORES
    zb = 32
    assert rps % zb == 0

    mesh = plsc.VectorSubcoreMesh(core_axis_name="c", subcore_axis_name="s")

    out_type = jax.ShapeDtypeStruct((N_CORES, n_acc, d), jnp.float32)
    if with_counts:
        out_type = [out_type,
                    jax.ShapeDtypeStruct((N_CORES * n_acc,), jnp.float32)]
    scratch = [
        pltpu.VMEM_SHARED((n_acc, d), jnp.float32),
        pltpu.VMEM((BATCH, CHUNK), jnp.int32),
        pltpu.VMEM((BATCH, CHUNK), jnp.int32),
        pltpu.VMEM((CHUNK, d), jnp.float32),
        pltpu.VMEM((CHUNK, d), jnp.float32),
        pltpu.VMEM((zb, d), jnp.float32),
        pltpu.SemaphoreType.DMA,
        pltpu.SemaphoreType.DMA,
        pltpu.SemaphoreType.DMA,
        pltpu.SemaphoreType.DMA,
    ]
    if with_counts:
        scratch += [
            pltpu.VMEM_SHARED((n_acc,), jnp.float32),
            pltpu.VMEM((CHUNK,), jnp.float32),
            pltpu.SemaphoreType.DMA,
        ]

    def body(x_hbm, src_hbm, dst_hbm, *refs):
        if with_counts:
            (sum_hbm, cnt_hbm, acc_sh, srcb_v, dstb_v, rows0_v, rows1_v,
             zrow_v, gsem0, gsem1, ssem0, ssem1, cnt_sh, ones_v, csem) = refs
        else:
            (sum_hbm, acc_sh, srcb_v, dstb_v, rows0_v, rows1_v,
             zrow_v, gsem0, gsem1, ssem0, ssem1) = refs
        cid = lax.axis_index("c")
        sid = lax.axis_index("s")
        wid = cid * N_SUBCORES + sid
        r0 = sid * rps
        rows = (rows0_v, rows1_v)
        gsem = (gsem0, gsem1)
        ssem = (ssem0, ssem1)

        @pl.loop(0, zb)
        def _(r):
            @pl.loop(0, d, step=LANES)
            def _(c):
                zrow_v[r, pl.ds(c, LANES)] = jnp.zeros((LANES,), jnp.float32)

        @pl.loop(0, rps, step=zb)
        def _(rr):
            pltpu.sync_copy(zrow_v, acc_sh.at[pl.ds(r0 + rr, zb)])

        if with_counts:
            @pl.loop(0, CHUNK, step=LANES)
            def _(r):
                ones_v[pl.ds(r, LANES)] = jnp.zeros((LANES,), jnp.float32)

            @pl.loop(0, rps, step=CHUNK)
            def _(rr):
                pltpu.sync_copy(ones_v, cnt_sh.at[pl.ds(r0 + rr, CHUNK)])

            @pl.loop(0, CHUNK, step=LANES)
            def _(r):
                ones_v[pl.ds(r, LANES)] = jnp.ones((LANES,), jnp.float32)

        plsc.subcore_barrier()

        nb = jnp.where(cid == 0, b0, b1)
        chunk0 = jnp.where(cid == 0, sid * b0,
                           N_SUBCORES * b0 + sid * b1) * BATCH

        @pl.loop(0, nb)
        def _(t):
            cbase = chunk0 + t * BATCH
            pltpu.sync_copy(src_hbm.at[pl.ds(cbase, BATCH)], srcb_v)
            pltpu.sync_copy(dst_hbm.at[pl.ds(cbase, BATCH)], dstb_v)
            ga = [
                pltpu.async_copy(x_hbm.at[srcb_v.at[0]], rows[0], gsem[0]),
                pltpu.async_copy(x_hbm.at[srcb_v.at[1]], rows[1], gsem[1]),
            ]
            sc = [None, None]
            cc = []
            for j in range(BATCH):
                b = j % 2
                ga[b].wait()
                sc[b] = pltpu.async_copy(rows[b], acc_sh.at[dstb_v.at[j]],
                                         ssem[b], add=True)
                if with_counts:
                    cc.append(pltpu.async_copy(
                        ones_v, cnt_sh.at[dstb_v.at[j]], csem, add=True))
                if 1 <= j < BATCH - 1:
                    sc[1 - b].wait()
                    ga[1 - b] = pltpu.async_copy(
                        x_hbm.at[srcb_v.at[j + 1]], rows[1 - b], gsem[1 - b])
            sc[0].wait()
            sc[1].wait()
            for h in cc:
                h.wait()

        plsc.subcore_barrier()

        pltpu.sync_copy(acc_sh.at[pl.ds(r0, rps)],
                        sum_hbm.at[cid, pl.ds(r0, rps)])
        if with_counts:
            pltpu.sync_copy(cnt_sh.at[pl.ds(r0, rps)],
                            cnt_hbm.at[pl.ds(cid * n_acc + r0, rps)])

    return pl.kernel(body, out_type=out_type, mesh=mesh,
                     scratch_types=scratch)


@functools.lru_cache(maxsize=None)
def _make_counts(n_nodes, n_chunks):
    assert n_chunks % (NW * BATCH) == 0
    n_batches = n_chunks // (NW * BATCH)
    n_acc = _acc_rows(n_nodes)
    rps = n_acc // N_SUBCORES

    mesh = plsc.VectorSubcoreMesh(core_axis_name="c", subcore_axis_name="s")

    out_type = jax.ShapeDtypeStruct((N_CORES * n_acc,), jnp.float32)
    scratch = [
        pltpu.VMEM_SHARED((n_acc,), jnp.float32),
        pltpu.VMEM((BATCH, CHUNK), jnp.int32),
        pltpu.VMEM((CHUNK,), jnp.float32),
        pltpu.SemaphoreType.DMA,
    ]

    def body(dst_hbm, cnt_hbm, cnt_sh, dstb_v, ones_v, sem):
        cid = lax.axis_index("c")
        sid = lax.axis_index("s")
        wid = cid * N_SUBCORES + sid
        r0 = sid * rps

        @pl.loop(0, CHUNK, step=LANES)
        def _(r):
            ones_v[pl.ds(r, LANES)] = jnp.zeros((LANES,), jnp.float32)

        @pl.loop(0, rps, step=CHUNK)
        def _(rr):
            pltpu.sync_copy(ones_v, cnt_sh.at[pl.ds(r0 + rr, CHUNK)])

        @pl.loop(0, CHUNK, step=LANES)
        def _(r):
            ones_v[pl.ds(r, LANES)] = jnp.ones((LANES,), jnp.float32)

        plsc.subcore_barrier()

        chunk0 = wid * (n_batches * BATCH)

        @pl.loop(0, n_batches)
        def _(t):
            pltpu.sync_copy(dst_hbm.at[pl.ds(chunk0 + t * BATCH, BATCH)],
                            dstb_v)
            for j in range(BATCH):
                pltpu.sync_copy(ones_v, cnt_sh.at[dstb_v.at[j]], add=True)

        plsc.subcore_barrier()
        pltpu.sync_copy(cnt_sh.at[pl.ds(r0, rps)],
                        cnt_hbm.at[pl.ds(cid * n_acc + r0, rps)])

    return pl.kernel(body, out_type=out_type, mesh=mesh,
                     scratch_types=scratch)


@functools.lru_cache(maxsize=None)
def _make_layer(n_acc, d, relu, bn):
    assert n_acc % bn == 0
    grid = (n_acc // bn,)

    def body(s_ref, c_ref, x_ref, wl_ref, wr_ref, b_ref, o_ref):
        s = s_ref[0] + s_ref[1]
        cnt = c_ref[:, 0:1] + c_ref[:, 1:2]
        mean = s / jnp.maximum(cnt, 1.0)
        acc = jnp.dot(mean, wl_ref[...], preferred_element_type=jnp.float32)
        acc = acc + jnp.dot(x_ref[...], wr_ref[...],
                            preferred_element_type=jnp.float32)
        acc = acc + b_ref[...]
        if relu:
            acc = jnp.maximum(acc, 0.0)
        o_ref[...] = acc

    return pl.pallas_call(
        body,
        grid=grid,
        in_specs=[
            pl.BlockSpec((N_CORES, bn, d), lambda i: (0, i, 0)),
            pl.BlockSpec((bn, N_CORES), lambda i: (i, 0)),
            pl.BlockSpec((bn, d), lambda i: (i, 0)),
            pl.BlockSpec((d, d), lambda i: (0, 0)),
            pl.BlockSpec((d, d), lambda i: (0, 0)),
            pl.BlockSpec((1, d), lambda i: (0, 0)),
        ],
        out_specs=pl.BlockSpec((bn, d), lambda i: (i, 0)),
        out_shape=jax.ShapeDtypeStruct((n_acc, d), jnp.float32),
    )


def kernel(x, edge_index, W1_l, b1_l, W1_r, b1_r, W2_l, b2_l, W2_r, b2_r):
    n, d = x.shape
    src = edge_index[0].astype(jnp.int32)
    dst = edge_index[1].astype(jnp.int32)
    n_acc = _acc_rows(n)
    src2d, dst2d = _pad_edges(src, dst, n)
    n_chunks = src2d.shape[0]
    x_p = jnp.pad(x, ((0, n_acc - n), (0, 0)))

    agg = _make_agg(n, n_chunks, d, _AGG_SPLIT)
    sums1, cnts = _make_agg(n, n_chunks, d, _AGG_SPLIT, True)(
        x, src2d, dst2d)
    cpair = cnts.reshape(N_CORES, n_acc).T
    h = _make_layer(n_acc, d, True, 1024)(
        sums1, cpair, x_p, W1_l, W1_r, (b1_l + b1_r).reshape(1, d))
    sums2 = agg(h, src2d, dst2d)
    out = _make_layer(n_acc, d, False, 1024)(
        sums2, cpair, h, W2_l, W2_r, (b2_l + b2_r).reshape(1, d))
    return out[:n]
